# baseline (speedup 1.0000x reference)
import functools

import jax
import jax.numpy as jnp
from jax import lax
from jax.experimental import pallas as pl
from jax.experimental.pallas import tpu as pltpu
from jax.experimental.pallas import tpu_sc as plsc

MAX_LENGTH = 1000

NUM_CORES = 2
NUM_SUBCORES = 16
NUM_WORKERS = NUM_CORES * NUM_SUBCORES


def _make_sc_kernel(seq_len: int, dim: int, dtype):
    j_per_worker = seq_len // NUM_WORKERS
    mesh = plsc.VectorSubcoreMesh(
        core_axis_name="c", subcore_axis_name="s",
        num_cores=NUM_CORES, num_subcores=NUM_SUBCORES,
    )

    @functools.partial(
        pl.kernel,
        out_type=jax.ShapeDtypeStruct((seq_len, seq_len, dim), dtype),
        mesh=mesh,
        compiler_params=pltpu.CompilerParams(use_tc_tiling_on_sc=False),
    )
    def sc_copy(w_hbm, out_hbm):
        wid = lax.axis_index("s") * NUM_CORES + lax.axis_index("c")
        j0 = wid * j_per_worker
        for t in range(j_per_worker):
            j = j0 + t
            pltpu.sync_copy(
                w_hbm.at[pl.ds(MAX_LENGTH - j, seq_len), :],
                out_hbm.at[:, j, :],
            )

    return sc_copy


def kernel(hidden_states, pe_k_weight):
    seq_len = hidden_states.shape[1]
    dim = pe_k_weight.shape[1]
    return _make_sc_kernel(seq_len, dim, pe_k_weight.dtype)(pe_k_weight)
